# baseline (speedup 1.0000x reference)
import jax
import jax.numpy as jnp
from jax import lax
from jax.experimental import pallas as pl
from jax.experimental.pallas import tpu as pltpu
from jax.experimental.pallas import tpu_sc as plsc

_N = 100000
_G = 256
_NC = 2
_NS = 16
_NW = _NC * _NS
_CHUNK = 3136
_VECS = _CHUNK // 16
_SKIP = (_NW * _CHUNK - _N) // 16
_ROW = _G + 1
_ACC = 16 * _ROW
_EPS = 1e-8

_sc_mesh = plsc.VectorSubcoreMesh(core_axis_name="c", subcore_axis_name="s")


def _sc_body(np_hbm, idx_hbm, out_hbm, np_v, idx_v, acc_v, red_v, sem0, sem1):
    c = lax.axis_index("c")
    s = lax.axis_index("s")
    wid = s * _NC + c
    base = jnp.minimum(wid * _CHUNK, _N - _CHUNK)
    cp0 = pltpu.async_copy(np_hbm.at[pl.ds(base, _CHUNK)], np_v, sem0)
    cp1 = pltpu.async_copy(idx_hbm.at[pl.ds(base, _CHUNK)], idx_v, sem1)
    zero = jnp.zeros((16,), jnp.float32)

    @plsc.parallel_loop(0, _ACC // 16, step=1, unroll=8)
    def _(j):
        acc_v[pl.ds(j * 16, 16)] = zero

    cp0.wait()
    cp1.wait()

    @pl.when(wid == _NW - 1)
    def _():
        dump = jnp.full((16,), _G, jnp.int32)

        @plsc.parallel_loop(0, _SKIP, step=1, unroll=2)
        def _(j):
            idx_v[pl.ds(j * 16, 16)] = dump

    lane_off = lax.iota(jnp.int32, 16) * _ROW

    @plsc.parallel_loop(0, _VECS, step=1, unroll=4)
    def _(i):
        o = i * 16
        v = np_v[pl.ds(o, 16)]
        raw = jnp.exp(v + _EPS) - 1.0
        addr = idx_v[pl.ds(o, 16)] + lane_off
        plsc.addupdate_scatter(acc_v, [addr], raw)

    @plsc.parallel_loop(0, _G // 16, step=1, unroll=1)
    def _(g):
        o = g * 16
        t = acc_v[pl.ds(o, 16)]
        for l in range(1, 16):
            t = t + acc_v[pl.ds(l * _ROW + o, 16)]
        red_v[pl.ds(o, 16)] = t

    pltpu.sync_copy(red_v, out_hbm.at[wid])


_sc_pass = pl.kernel(
    _sc_body,
    out_type=jax.ShapeDtypeStruct((_NW, _G), jnp.float32),
    mesh=_sc_mesh,
    scratch_types=[
        pltpu.VMEM((_CHUNK,), jnp.float32),
        pltpu.VMEM((_CHUNK,), jnp.int32),
        pltpu.VMEM((_ACC,), jnp.float32),
        pltpu.VMEM((_G,), jnp.float32),
        pltpu.SemaphoreType.DMA,
        pltpu.SemaphoreType.DMA,
    ],
    compiler_params=pltpu.CompilerParams(needs_layout_passes=False),
)


def _mse_body(np_ref, nt_ref, gp_ref, gt_ref, sq_ref, gl_ref):
    d = np_ref[...] - nt_ref[...]
    sq_ref[...] = jnp.broadcast_to(jnp.sum(d * d), (1,))
    g = gp_ref[...] - gt_ref[...]
    gl_ref[...] = jnp.broadcast_to(jnp.sum(g * g), (1,))


_tc_mse = pl.pallas_call(
    _mse_body,
    out_shape=[jax.ShapeDtypeStruct((1,), jnp.float32)] * 2,
    out_specs=[pl.BlockSpec(memory_space=pltpu.VMEM)] * 2,
)


def _tc_combine_body(part_ref, gp_ref, sq_ref, gl_ref, ec_ref, total_ref,
                     node_ref, glob_ref, cons_ref):
    part = part_ref[...]
    seg = jnp.sum(part, axis=0)
    gp = gp_ref[...]
    nsl = jnp.log1p(seg + _EPS)
    cons = jnp.sum((nsl - gp) ** 2) / _G
    gl = jnp.sum(gl_ref[...]) / _G
    nl = jnp.sum(sq_ref[...]) / _N
    flag = ec_ref[0] != 0
    total_ref[0] = nl + gl + jnp.where(flag, 0.1 * cons, 0.0)
    node_ref[0] = nl
    glob_ref[0] = gl
    cons_ref[0] = jnp.where(flag, cons, 0.0)


_tc_combine = pl.pallas_call(
    _tc_combine_body,
    out_shape=[jax.ShapeDtypeStruct((1,), jnp.float32)] * 4,
    in_specs=[
        pl.BlockSpec(memory_space=pltpu.VMEM),
        pl.BlockSpec(memory_space=pltpu.VMEM),
        pl.BlockSpec(memory_space=pltpu.VMEM),
        pl.BlockSpec(memory_space=pltpu.VMEM),
        pl.BlockSpec(memory_space=pltpu.SMEM),
    ],
    out_specs=[pl.BlockSpec(memory_space=pltpu.SMEM)] * 4,
)


def kernel(node_pred, node_target, global_pred, global_target, batch_idx,
           enable_consistency=1):
    idx = batch_idx.astype(jnp.int32)
    ec = jnp.asarray(enable_consistency, jnp.int32).reshape(1)
    part = _sc_pass(node_pred, idx)
    sq, gl = _tc_mse(node_pred, node_target, global_pred, global_target)
    total, node, glob, cons = _tc_combine(part, global_pred, sq, gl, ec)
    return (total[0], node[0], glob[0], cons[0])


# speedup vs baseline: 1.0039x; 1.0039x over previous
import jax
import jax.numpy as jnp
from jax import lax
from jax.experimental import pallas as pl
from jax.experimental.pallas import tpu as pltpu
from jax.experimental.pallas import tpu_sc as plsc

_N = 100000
_G = 256
_NC = 2
_NS = 16
_NW = _NC * _NS
_CHUNK = 3136
_VECS = _CHUNK // 16
_SKIP = (_NW * _CHUNK - _N) // 16
_ROW = _G + 1
_ACC = 16 * _ROW
_EPS = 1e-8

_sc_mesh = plsc.VectorSubcoreMesh(core_axis_name="c", subcore_axis_name="s")


def _sc_body(np_hbm, idx_hbm, out_hbm, np_v, idx_v, acc_v, red_v, sem0, sem1):
    c = lax.axis_index("c")
    s = lax.axis_index("s")
    wid = s * _NC + c
    base = jnp.minimum(wid * _CHUNK, _N - _CHUNK)
    cp0 = pltpu.async_copy(np_hbm.at[pl.ds(base, _CHUNK)], np_v, sem0)
    cp1 = pltpu.async_copy(idx_hbm.at[pl.ds(base, _CHUNK)], idx_v, sem1)
    zero = jnp.zeros((16,), jnp.float32)

    @plsc.parallel_loop(0, _ACC // 16, step=1, unroll=8)
    def _(j):
        acc_v[pl.ds(j * 16, 16)] = zero

    cp0.wait()
    cp1.wait()

    @pl.when(wid == _NW - 1)
    def _():
        dump = jnp.full((16,), _G, jnp.int32)

        @plsc.parallel_loop(0, _SKIP, step=1, unroll=2)
        def _(j):
            idx_v[pl.ds(j * 16, 16)] = dump

    lane_off = lax.iota(jnp.int32, 16) * _ROW

    @plsc.parallel_loop(0, _VECS, step=1, unroll=7)
    def _(i):
        o = i * 16
        v = np_v[pl.ds(o, 16)]
        raw = jnp.exp(v + _EPS) - 1.0
        addr = idx_v[pl.ds(o, 16)] + lane_off
        plsc.addupdate_scatter(acc_v, [addr], raw)

    @plsc.parallel_loop(0, _G // 16, step=1, unroll=1)
    def _(g):
        o = g * 16
        t = acc_v[pl.ds(o, 16)]
        for l in range(1, 16):
            t = t + acc_v[pl.ds(l * _ROW + o, 16)]
        red_v[pl.ds(o, 16)] = t

    pltpu.sync_copy(red_v, out_hbm.at[wid])


_sc_pass = pl.kernel(
    _sc_body,
    out_type=jax.ShapeDtypeStruct((_NW, _G), jnp.float32),
    mesh=_sc_mesh,
    scratch_types=[
        pltpu.VMEM((_CHUNK,), jnp.float32),
        pltpu.VMEM((_CHUNK,), jnp.int32),
        pltpu.VMEM((_ACC,), jnp.float32),
        pltpu.VMEM((_G,), jnp.float32),
        pltpu.SemaphoreType.DMA,
        pltpu.SemaphoreType.DMA,
    ],
    compiler_params=pltpu.CompilerParams(needs_layout_passes=False),
)


def _mse_body(np_ref, nt_ref, gp_ref, gt_ref, sq_ref, gl_ref):
    d = np_ref[...] - nt_ref[...]
    sq_ref[...] = jnp.broadcast_to(jnp.sum(d * d), (1,))
    g = gp_ref[...] - gt_ref[...]
    gl_ref[...] = jnp.broadcast_to(jnp.sum(g * g), (1,))


_tc_mse = pl.pallas_call(
    _mse_body,
    out_shape=[jax.ShapeDtypeStruct((1,), jnp.float32)] * 2,
    out_specs=[pl.BlockSpec(memory_space=pltpu.VMEM)] * 2,
)


def _tc_combine_body(part_ref, gp_ref, sq_ref, gl_ref, ec_ref, total_ref,
                     node_ref, glob_ref, cons_ref):
    part = part_ref[...]
    seg = jnp.sum(part, axis=0)
    gp = gp_ref[...]
    nsl = jnp.log1p(seg + _EPS)
    cons = jnp.sum((nsl - gp) ** 2) / _G
    gl = jnp.sum(gl_ref[...]) / _G
    nl = jnp.sum(sq_ref[...]) / _N
    flag = ec_ref[0] != 0
    total_ref[0] = nl + gl + jnp.where(flag, 0.1 * cons, 0.0)
    node_ref[0] = nl
    glob_ref[0] = gl
    cons_ref[0] = jnp.where(flag, cons, 0.0)


_tc_combine = pl.pallas_call(
    _tc_combine_body,
    out_shape=[jax.ShapeDtypeStruct((1,), jnp.float32)] * 4,
    in_specs=[
        pl.BlockSpec(memory_space=pltpu.VMEM),
        pl.BlockSpec(memory_space=pltpu.VMEM),
        pl.BlockSpec(memory_space=pltpu.VMEM),
        pl.BlockSpec(memory_space=pltpu.VMEM),
        pl.BlockSpec(memory_space=pltpu.SMEM),
    ],
    out_specs=[pl.BlockSpec(memory_space=pltpu.SMEM)] * 4,
)


def kernel(node_pred, node_target, global_pred, global_target, batch_idx,
           enable_consistency=1):
    idx = batch_idx.astype(jnp.int32)
    ec = jnp.asarray(enable_consistency, jnp.int32).reshape(1)
    part = _sc_pass(node_pred, idx)
    sq, gl = _tc_mse(node_pred, node_target, global_pred, global_target)
    total, node, glob, cons = _tc_combine(part, global_pred, sq, gl, ec)
    return (total[0], node[0], glob[0], cons[0])
